# Initial kernel scaffold; baseline (speedup 1.0000x reference)
import functools

import jax
import jax.numpy as jnp
from jax import lax
from jax.experimental import pallas as pl
from jax.experimental.pallas import tpu as pltpu
from jax.experimental.pallas import tpu_sc as plsc

_N = 10000
_E = 320000
_D = 128
_H = 128
_M = 3
_DEPTH = 3
_H3 = _H // 3
_H9 = _H3 // 3

_NC = 2
_NS = 16
_NW = _NC * _NS
_EPT = _E // _NW
_K = 80
_NCHUNK = _EPT // _K
_RPT = _N // _NS

_HI = jax.lax.Precision.HIGHEST


@functools.partial(
    pl.kernel,
    out_type=jax.ShapeDtypeStruct((_M * 2 * _N, _H), jnp.float32),
    mesh=plsc.VectorSubcoreMesh(core_axis_name="c", subcore_axis_name="s"),
    scratch_types=[
        pltpu.VMEM((_NCHUNK, _K), jnp.int32),
        pltpu.VMEM((_NCHUNK, _K), jnp.int32),
        pltpu.VMEM((_K, _H), jnp.float32),
        pltpu.VMEM_SHARED((_N, _H), jnp.float32),
        pltpu.SemaphoreType.DMA,
    ],
)
def _sc_segsum(h_hbm, src_hbm, dst_hbm, zeros_hbm, agg_hbm,
               src_v, dst_v, rows_v, acc, sem):
    c = lax.axis_index("c")
    s = lax.axis_index("s")
    wid = c * _NS + s
    pltpu.sync_copy(dst_hbm.at[wid], dst_v)
    for m in range(_M):
        pltpu.sync_copy(src_hbm.at[m * _NW + wid], src_v)
        pltpu.sync_copy(zeros_hbm.at[pl.ds(s * _RPT, _RPT)],
                        acc.at[pl.ds(s * _RPT, _RPT)])
        plsc.subcore_barrier()

        def chunk(j, carry):
            pltpu.async_copy(h_hbm.at[src_v.at[j]], rows_v, sem).wait()
            pltpu.sync_copy(rows_v, acc.at[dst_v.at[j]], add=True)
            return carry

        lax.fori_loop(0, _NCHUNK, chunk, 0)
        plsc.subcore_barrier()
        row0 = (2 * m + c) * _N + s * _RPT
        pltpu.sync_copy(acc.at[pl.ds(s * _RPT, _RPT)],
                        agg_hbm.at[pl.ds(row0, _RPT)])


_BN = 1000
_NB = _N // _BN


def _h0_body(x_ref, wi_ref, out_ref):
    x = x_ref[...]
    for m in range(_M):
        out_ref[m] = jnp.maximum(lax.dot(x, wi_ref[m], precision=_HI), 0.0)


_h0_call = pl.pallas_call(
    _h0_body,
    grid=(_NB,),
    in_specs=[
        pl.BlockSpec((_BN, _D), lambda i: (i, 0)),
        pl.BlockSpec((_M, _D, _H), lambda i: (0, 0, 0)),
    ],
    out_specs=pl.BlockSpec((_M, _BN, _H), lambda i: (0, i, 0)),
    out_shape=jax.ShapeDtypeStruct((_M, _N, _H), jnp.float32),
)


def _upd_body(h0_ref, agg_ref, wh_ref, out_ref):
    for m in range(_M):
        a = agg_ref[m, 0] + agg_ref[m, 1]
        out_ref[m] = jnp.maximum(
            h0_ref[m] + lax.dot(a, wh_ref[m], precision=_HI), 0.0)


_upd_call = pl.pallas_call(
    _upd_body,
    grid=(_NB,),
    in_specs=[
        pl.BlockSpec((_M, _BN, _H), lambda i: (0, i, 0)),
        pl.BlockSpec((_M, 2, _BN, _H), lambda i: (0, 0, i, 0)),
        pl.BlockSpec((_M, _H, _H), lambda i: (0, 0, 0)),
    ],
    out_specs=pl.BlockSpec((_M, _BN, _H), lambda i: (0, i, 0)),
    out_shape=jax.ShapeDtypeStruct((_M, _N, _H), jnp.float32),
)


def _readout_body(x_ref, h_ref, wo_ref, bo_ref, w1_ref, b1_ref, w2_ref,
                  b2_ref, cw1_ref, cb1_ref, cw2_ref, cb2_ref, cw3_ref,
                  cb3_ref, out_ref, acc_ref):
    i = pl.program_id(0)

    @pl.when(i == 0)
    def _():
        acc_ref[...] = jnp.zeros_like(acc_ref)

    x = x_ref[...]
    for m in range(_M):
        ah = jnp.maximum(
            lax.dot(x, wo_ref[m, :_D, :], precision=_HI)
            + lax.dot(h_ref[m], wo_ref[m, _D:, :], precision=_HI)
            + bo_ref[m][None, :], 0.0)
        acc_ref[m, :] = acc_ref[m, :] + jnp.sum(ah, axis=0)

    @pl.when(i == _NB - 1)
    def _():
        total = 0.0
        for m in range(_M):
            e = acc_ref[m, :] * (1.0 / _N)
            t = jnp.maximum(
                jnp.sum(e[:, None] * w1_ref[m], axis=0) + b1_ref[m], 0.0)
            temp = jnp.sum(t * w2_ref[m]) + b2_ref[m]
            z = jnp.maximum(
                jnp.sum(e[:, None] * cw1_ref[m], axis=0) + cb1_ref[m], 0.0)
            z2 = jnp.maximum(
                jnp.sum(z[:, None] * cw2_ref[m], axis=0) + cb2_ref[m], 0.0)
            coef = jnp.sum(z2 * cw3_ref[m]) + cb3_ref[m]
            total = total + temp * coef
        out_ref[0, 0] = total


_readout_call = pl.pallas_call(
    _readout_body,
    grid=(_NB,),
    in_specs=[
        pl.BlockSpec((_BN, _D), lambda i: (i, 0)),
        pl.BlockSpec((_M, _BN, _H), lambda i: (0, i, 0)),
        pl.BlockSpec((_M, _D + _H, _H), lambda i: (0, 0, 0)),
        pl.BlockSpec((_M, _H), lambda i: (0, 0)),
        pl.BlockSpec((_M, _H, _H), lambda i: (0, 0, 0)),
        pl.BlockSpec((_M, _H), lambda i: (0, 0)),
        pl.BlockSpec((_M, _H), lambda i: (0, 0)),
        pl.BlockSpec((_M,), lambda i: (0,)),
        pl.BlockSpec((_M, _H, _H3), lambda i: (0, 0, 0)),
        pl.BlockSpec((_M, _H3), lambda i: (0, 0)),
        pl.BlockSpec((_M, _H3, _H9), lambda i: (0, 0, 0)),
        pl.BlockSpec((_M, _H9), lambda i: (0, 0)),
        pl.BlockSpec((_M, _H9), lambda i: (0, 0)),
        pl.BlockSpec((_M,), lambda i: (0,)),
    ],
    out_specs=pl.BlockSpec((1, 1), lambda i: (0, 0)),
    out_shape=jax.ShapeDtypeStruct((1, 1), jnp.float32),
    scratch_shapes=[pltpu.VMEM((_M, _H), jnp.float32)],
)


def kernel(x, edge_index, W_i, W_h, W_o, b_o, ffn_W1, ffn_b1, ffn_W2, ffn_b2,
           c_W1, c_b1, c_W2, c_b2, c_W3, c_b3):
    src = edge_index[0]
    dst = edge_index[1]
    offs = (jnp.arange(_M, dtype=jnp.int32) * _N)[:, None]
    src_m = (src[None, :] + offs).reshape(_M * _NW, _NCHUNK, _K)
    dst_r = dst.reshape(_NW, _NCHUNK, _K)
    zeros = jnp.zeros((_N, _H), jnp.float32)

    h0 = _h0_call(x, W_i)
    h = h0
    for _ in range(_DEPTH):
        agg_flat = _sc_segsum(h.reshape(_M * _N, _H), src_m, dst_r, zeros)
        agg = agg_flat.reshape(_M, 2, _N, _H)
        h = _upd_call(h0, agg, W_h)
    out = _readout_call(x, h, W_o, b_o, ffn_W1, ffn_b1, ffn_W2[..., 0],
                        ffn_b2[..., 0], c_W1, c_b1, c_W2, c_b2, c_W3[..., 0],
                        c_b3[..., 0])
    return out


# speedup vs baseline: 5.2533x; 5.2533x over previous
import functools

import jax
import jax.numpy as jnp
from jax import lax
from jax.experimental import pallas as pl
from jax.experimental.pallas import tpu as pltpu
from jax.experimental.pallas import tpu_sc as plsc

_N = 10000
_E = 320000
_D = 128
_H = 128
_M = 3
_DEPTH = 3
_H3 = _H // 3
_H9 = _H3 // 3

_NC = 2
_NS = 16
_NW = _NC * _NS
_EPT = _E // _NW
_K = 80
_NCHUNK = _EPT // _K
_NPAD = 10240
_RPT = _NPAD // _NS

@functools.partial(
    pl.kernel,
    out_type=jax.ShapeDtypeStruct((_M * 2 * _NPAD, _H), jnp.float32),
    mesh=plsc.VectorSubcoreMesh(core_axis_name="c", subcore_axis_name="s"),
    scratch_types=[
        pltpu.VMEM((_NCHUNK, _K), jnp.int32),
        pltpu.VMEM((_NCHUNK, _K), jnp.int32),
        pltpu.VMEM((_K, _H), jnp.float32),
        pltpu.VMEM_SHARED((_NPAD, _H), jnp.float32),
        pltpu.SemaphoreType.DMA,
    ],
)
def _sc_segsum(h_hbm, src_hbm, dst_hbm, zeros_hbm, agg_hbm,
               src_v, dst_v, rows_v, acc, sem):
    c = lax.axis_index("c")
    s = lax.axis_index("s")
    wid = c * _NS + s
    pltpu.sync_copy(dst_hbm.at[wid], dst_v)
    for m in range(_M):
        pltpu.sync_copy(src_hbm.at[m * _NW + wid], src_v)
        pltpu.sync_copy(zeros_hbm, acc.at[pl.ds(s * _RPT, _RPT)])
        plsc.subcore_barrier()

        def chunk(j, carry):
            pltpu.async_copy(h_hbm.at[src_v.at[j]], rows_v, sem).wait()
            pltpu.sync_copy(rows_v, acc.at[dst_v.at[j]], add=True)
            return carry

        lax.fori_loop(0, _NCHUNK, chunk, 0)
        plsc.subcore_barrier()
        row0 = (2 * m + c) * _NPAD + s * _RPT
        pltpu.sync_copy(acc.at[pl.ds(s * _RPT, _RPT)],
                        agg_hbm.at[pl.ds(row0, _RPT)])


_BN = 1000
_NB = _N // _BN


def _h0_body(x_ref, wi_ref, out_ref):
    x = x_ref[...]
    for m in range(_M):
        out_ref[m] = jnp.maximum(lax.dot(x, wi_ref[m]), 0.0)


_h0_call = pl.pallas_call(
    _h0_body,
    grid=(_NB,),
    in_specs=[
        pl.BlockSpec((_BN, _D), lambda i: (i, 0)),
        pl.BlockSpec((_M, _D, _H), lambda i: (0, 0, 0)),
    ],
    out_specs=pl.BlockSpec((_M, _BN, _H), lambda i: (0, i, 0)),
    out_shape=jax.ShapeDtypeStruct((_M, _N, _H), jnp.float32),
)


def _upd_body(h0_ref, agg_ref, wh_ref, out_ref):
    for m in range(_M):
        a = agg_ref[m, 0] + agg_ref[m, 1]
        out_ref[m] = jnp.maximum(
            h0_ref[m] + lax.dot(a, wh_ref[m]), 0.0)


_upd_call = pl.pallas_call(
    _upd_body,
    grid=(_NB,),
    in_specs=[
        pl.BlockSpec((_M, _BN, _H), lambda i: (0, i, 0)),
        pl.BlockSpec((_M, 2, _BN, _H), lambda i: (0, 0, i, 0)),
        pl.BlockSpec((_M, _H, _H), lambda i: (0, 0, 0)),
    ],
    out_specs=pl.BlockSpec((_M, _BN, _H), lambda i: (0, i, 0)),
    out_shape=jax.ShapeDtypeStruct((_M, _N, _H), jnp.float32),
)


def _readout_body(x_ref, h_ref, wo_ref, bo_ref, w1_ref, b1_ref, w2_ref,
                  b2_ref, cw1_ref, cb1_ref, cw2_ref, cb2_ref, cw3_ref,
                  cb3_ref, out_ref, acc_ref):
    i = pl.program_id(0)

    @pl.when(i == 0)
    def _():
        acc_ref[...] = jnp.zeros_like(acc_ref)

    x = x_ref[...]
    for m in range(_M):
        ah = jnp.maximum(
            lax.dot(x, wo_ref[m, :_D, :])
            + lax.dot(h_ref[m], wo_ref[m, _D:, :])
            + bo_ref[m][None, :], 0.0)
        acc_ref[m, :] = acc_ref[m, :] + jnp.sum(ah, axis=0)

    @pl.when(i == _NB - 1)
    def _():
        def rb(v):
            return v.astype(jnp.bfloat16).astype(jnp.float32)

        total = 0.0
        for m in range(_M):
            e = rb(acc_ref[m, :] * (1.0 / _N))
            t = jnp.maximum(
                jnp.sum(e[:, None] * rb(w1_ref[m]), axis=0) + b1_ref[m], 0.0)
            temp = jnp.sum(rb(t) * rb(w2_ref[m])) + b2_ref[m]
            z = jnp.maximum(
                jnp.sum(e[:, None] * rb(cw1_ref[m]), axis=0) + cb1_ref[m], 0.0)
            z2 = jnp.maximum(
                jnp.sum(rb(z)[:, None] * rb(cw2_ref[m]), axis=0) + cb2_ref[m], 0.0)
            coef = jnp.sum(rb(z2) * rb(cw3_ref[m])) + cb3_ref[m]
            total = total + temp * coef
        out_ref[...] = jnp.reshape(total, (1, 1))


_readout_call = pl.pallas_call(
    _readout_body,
    grid=(_NB,),
    in_specs=[
        pl.BlockSpec((_BN, _D), lambda i: (i, 0)),
        pl.BlockSpec((_M, _BN, _H), lambda i: (0, i, 0)),
        pl.BlockSpec((_M, _D + _H, _H), lambda i: (0, 0, 0)),
        pl.BlockSpec((_M, _H), lambda i: (0, 0)),
        pl.BlockSpec((_M, _H, _H), lambda i: (0, 0, 0)),
        pl.BlockSpec((_M, _H), lambda i: (0, 0)),
        pl.BlockSpec((_M, _H), lambda i: (0, 0)),
        pl.BlockSpec((_M,), lambda i: (0,)),
        pl.BlockSpec((_M, _H, _H3), lambda i: (0, 0, 0)),
        pl.BlockSpec((_M, _H3), lambda i: (0, 0)),
        pl.BlockSpec((_M, _H3, _H9), lambda i: (0, 0, 0)),
        pl.BlockSpec((_M, _H9), lambda i: (0, 0)),
        pl.BlockSpec((_M, _H9), lambda i: (0, 0)),
        pl.BlockSpec((_M,), lambda i: (0,)),
    ],
    out_specs=pl.BlockSpec((1, 1), lambda i: (0, 0)),
    out_shape=jax.ShapeDtypeStruct((1, 1), jnp.float32),
    scratch_shapes=[pltpu.VMEM((_M, _H), jnp.float32)],
)


def kernel(x, edge_index, W_i, W_h, W_o, b_o, ffn_W1, ffn_b1, ffn_W2, ffn_b2,
           c_W1, c_b1, c_W2, c_b2, c_W3, c_b3):
    src = edge_index[0]
    dst = edge_index[1]
    offs = (jnp.arange(_M, dtype=jnp.int32) * _N)[:, None]
    src_m = (src[None, :] + offs).reshape(_M * _NW, _NCHUNK, _K)
    dst_r = dst.reshape(_NW, _NCHUNK, _K)
    zeros = jnp.zeros((_RPT, _H), jnp.float32)

    h0 = _h0_call(x, W_i)
    h = h0
    for _ in range(_DEPTH):
        agg_flat = _sc_segsum(h.reshape(_M * _N, _H), src_m, dst_r, zeros)
        agg = agg_flat.reshape(_M, 2, _NPAD, _H)
        h = _upd_call(h0, agg, W_h)
    out = _readout_call(x, h, W_o, b_o, ffn_W1, ffn_b1, ffn_W2[..., 0],
                        ffn_b2[..., 0], c_W1, c_b1, c_W2, c_b2, c_W3[..., 0],
                        c_b3[..., 0])
    return out
